# baseline (speedup 1.0000x reference)
import functools

import jax
import jax.numpy as jnp
from jax import lax
from jax.experimental import pallas as pl
from jax.experimental.pallas import tpu as pltpu
from jax.experimental.pallas import tpu_sc as plsc

N_NODES = 10000
N_EDGES = 320000
D_FEAT = 128
HALF = 64
CHUNK = 128
REAL = CHUNK - 1
NSUB = 16
JT = 160
JH = JT // 4
JSTAGE = JH + 1
NCHUNK = NSUB * JT
NC_OUT = NSUB * JT + 1
E_PAD = NCHUNK * REAL
ROWS_PER_TILE = N_NODES // NSUB
ZROWS = 25

_mesh = plsc.VectorSubcoreMesh(core_axis_name="c", subcore_axis_name="s")


def _build(interpret=False):
    return functools.partial(
        pl.kernel,
        out_type=jax.ShapeDtypeStruct((2, N_NODES, HALF), jnp.float32),
        mesh=_mesh,
        scratch_types=[
            pltpu.VMEM_SHARED((N_NODES, HALF), jnp.float32),
            pltpu.VMEM_SHARED((N_NODES, HALF), jnp.float32),
            pltpu.VMEM((JSTAGE, 2, CHUNK), jnp.int32),
            pltpu.VMEM((JSTAGE, CHUNK), jnp.float32),
            pltpu.VMEM((CHUNK, HALF), jnp.float32),
            pltpu.VMEM((CHUNK, HALF), jnp.float32),
            pltpu.VMEM((ZROWS, HALF), jnp.float32),
            pltpu.SemaphoreType.DMA,
            pltpu.SemaphoreType.DMA,
            pltpu.SemaphoreType.DMA,
            pltpu.SemaphoreType.DMA,
        ],
        compiler_params=pltpu.CompilerParams(use_tc_tiling_on_sc=False,
                                             needs_layout_passes=False),
        interpret=interpret,
    )


def _spmm2_body(xs_hbm, eidx_hbm, evals_hbm, out_hbm,
                src_sh, acc_sh, eidx_v, vals_v, rows_a, rows_b, zero_v,
                gsem_a, gsem_b, ssem_a, ssem_b):
    c = lax.axis_index("c")
    s = lax.axis_index("s")
    r0 = s * ROWS_PER_TILE
    j0_tile = s * JT

    def zero_body(i, carry):
        for g in range(HALF // 16):
            zero_v[i, pl.ds(g * 16, 16)] = jnp.zeros((16,), jnp.float32)
        return carry
    lax.fori_loop(0, ZROWS, zero_body, 0)

    pltpu.sync_copy(xs_hbm.at[pl.ds(c * N_NODES + r0, ROWS_PER_TILE)],
                    src_sh.at[pl.ds(r0, ROWS_PER_TILE)])
    for z in range(ROWS_PER_TILE // ZROWS):
        pltpu.sync_copy(zero_v, acc_sh.at[pl.ds(r0 + z * ZROWS, ZROWS)])
    plsc.subcore_barrier()

    def run_layer(src, dst):
        def scale(j, rows_v):
            for g in range(HALF // 16):
                rows_v[0, pl.ds(g * 16, 16)] = jnp.zeros((16,), jnp.float32)

            @plsc.parallel_loop(1, CHUNK, unroll=4)
            def _(e):
                v = plsc.load_gather(
                    vals_v, [jnp.full((16,), j, jnp.int32),
                             jnp.full((16,), e, jnp.int32)])
                for g in range(HALF // 16):
                    sl = pl.ds(g * 16, 16)
                    rows_v[e, sl] = rows_v[e, sl] * v

        def gather(j, rows_v, sem):
            pltpu.async_copy(src.at[eidx_v.at[j, 1]], rows_v, sem)

        def gwait(rows_v, sem):
            pltpu.make_async_copy(src.at[eidx_v.at[0, 1]], rows_v, sem).wait()

        def scatter(j, rows_v, sem):
            pltpu.async_copy(rows_v, dst.at[eidx_v.at[j, 0]], sem, add=True)

        def swait(rows_v, sem):
            pltpu.make_async_copy(rows_v, dst.at[eidx_v.at[0, 0]], sem).wait()

        for half in range(4):
            pltpu.sync_copy(eidx_hbm.at[pl.ds(j0_tile + half * JH, JSTAGE)],
                            eidx_v)
            pltpu.sync_copy(evals_hbm.at[pl.ds(j0_tile + half * JH, JSTAGE)],
                            vals_v)
            gather(0, rows_a, gsem_a)
            gather(1, rows_b, gsem_b)
            gwait(rows_a, gsem_a)
            scale(0, rows_a)
            scatter(0, rows_a, ssem_a)
            gwait(rows_b, gsem_b)
            scale(1, rows_b)
            swait(rows_a, ssem_a)
            gather(2, rows_a, gsem_a)
            scatter(1, rows_b, ssem_b)

            def body(jj, carry):
                ja = 2 * jj
                jb = 2 * jj + 1
                swait(rows_b, ssem_b)
                gather(jb, rows_b, gsem_b)
                gwait(rows_a, gsem_a)
                scale(ja, rows_a)
                scatter(ja, rows_a, ssem_a)
                gwait(rows_b, gsem_b)
                scale(jb, rows_b)
                swait(rows_a, ssem_a)
                gather(ja + 2, rows_a, gsem_a)
                scatter(jb, rows_b, ssem_b)
                return carry
            lax.fori_loop(1, JH // 2, body, 0)
            gwait(rows_a, gsem_a)
            swait(rows_b, ssem_b)

    run_layer(src_sh, acc_sh)
    plsc.subcore_barrier()
    for z in range(ROWS_PER_TILE // ZROWS):
        pltpu.sync_copy(zero_v, src_sh.at[pl.ds(r0 + z * ZROWS, ZROWS)])
    plsc.subcore_barrier()
    run_layer(acc_sh, src_sh)
    plsc.subcore_barrier()

    pltpu.sync_copy(src_sh.at[pl.ds(r0, ROWS_PER_TILE)],
                    out_hbm.at[c, pl.ds(r0, ROWS_PER_TILE)])


_spmm2 = _build()(_spmm2_body)


def kernel(x, edge_index, values):
    xs = jnp.concatenate([x[:, :HALF], x[:, HALF:]], axis=0)
    pad = E_PAD - N_EDGES
    row = jnp.pad(edge_index[0], (0, pad)).reshape(NCHUNK, REAL)
    col = jnp.pad(edge_index[1], (0, pad)).reshape(NCHUNK, REAL)
    val = jnp.pad(values, (0, pad)).reshape(NCHUNK, REAL)
    zero_lane_i = jnp.zeros((NCHUNK, 1), jnp.int32)
    zero_lane_f = jnp.zeros((NCHUNK, 1), jnp.float32)
    row = jnp.concatenate([zero_lane_i, row], axis=1)[:, None, :]
    col = jnp.concatenate([zero_lane_i, col], axis=1)[:, None, :]
    eidx = jnp.concatenate([row, col], axis=1)
    evals = jnp.concatenate([zero_lane_f, val], axis=1)
    eidx = jnp.pad(eidx, ((0, NC_OUT - NCHUNK), (0, 0), (0, 0)))
    evals = jnp.pad(evals, ((0, NC_OUT - NCHUNK), (0, 0)))
    o = _spmm2(xs, eidx, evals)
    return jnp.concatenate([o[0], o[1]], axis=1)


# speedup vs baseline: 1.0542x; 1.0542x over previous
import functools

import jax
import jax.numpy as jnp
from jax import lax
from jax.experimental import pallas as pl
from jax.experimental.pallas import tpu as pltpu
from jax.experimental.pallas import tpu_sc as plsc

N_NODES = 10000
N_EDGES = 320000
D_FEAT = 128
HALF = 64
CHUNK = 128
REAL = CHUNK - 1
NSUB = 16
JT = 160
JH = JT // 4
JSTAGE = JH + 1
NCHUNK = NSUB * JT
NC_OUT = NSUB * JT + 1
E_PAD = NCHUNK * REAL
ROWS_PER_TILE = N_NODES // NSUB
ZROWS = 25

_mesh = plsc.VectorSubcoreMesh(core_axis_name="c", subcore_axis_name="s")


def _build(interpret=False):
    return functools.partial(
        pl.kernel,
        out_type=jax.ShapeDtypeStruct((2, N_NODES, HALF), jnp.float32),
        mesh=_mesh,
        scratch_types=[
            pltpu.VMEM_SHARED((N_NODES, HALF), jnp.float32),
            pltpu.VMEM_SHARED((N_NODES, HALF), jnp.float32),
            pltpu.VMEM((JSTAGE, 2, CHUNK), jnp.int32),
            pltpu.VMEM((JSTAGE, CHUNK), jnp.float32),
            pltpu.VMEM((CHUNK, HALF), jnp.float32),
            pltpu.VMEM((CHUNK, HALF), jnp.float32),
            pltpu.VMEM((ZROWS, HALF), jnp.float32),
            pltpu.SemaphoreType.DMA,
            pltpu.SemaphoreType.DMA,
        ],
        compiler_params=pltpu.CompilerParams(use_tc_tiling_on_sc=False,
                                             needs_layout_passes=False),
        interpret=interpret,
    )


def _spmm2_body(xs_hbm, eidx_hbm, evals_hbm, out_hbm,
                src_sh, acc_sh, eidx_v, vals_v, rows_a, rows_b, zero_v,
                gsem_a, gsem_b):
    c = lax.axis_index("c")
    s = lax.axis_index("s")
    r0 = s * ROWS_PER_TILE
    j0_tile = s * JT

    def zero_body(i, carry):
        for g in range(HALF // 16):
            zero_v[i, pl.ds(g * 16, 16)] = jnp.zeros((16,), jnp.float32)
        return carry
    lax.fori_loop(0, ZROWS, zero_body, 0)

    pltpu.sync_copy(xs_hbm.at[pl.ds(c * N_NODES + r0, ROWS_PER_TILE)],
                    src_sh.at[pl.ds(r0, ROWS_PER_TILE)])
    for z in range(ROWS_PER_TILE // ZROWS):
        pltpu.sync_copy(zero_v, acc_sh.at[pl.ds(r0 + z * ZROWS, ZROWS)])
    plsc.subcore_barrier()

    def run_layer(src, dst):
        def process(j, rows_v):
            for g in range(HALF // 16):
                rows_v[0, pl.ds(g * 16, 16)] = jnp.zeros((16,), jnp.float32)

            @plsc.parallel_loop(1, CHUNK, unroll=4)
            def _(e):
                v = plsc.load_gather(
                    vals_v, [jnp.full((16,), j, jnp.int32),
                             jnp.full((16,), e, jnp.int32)])
                for g in range(HALF // 16):
                    sl = pl.ds(g * 16, 16)
                    rows_v[e, sl] = rows_v[e, sl] * v
            pltpu.sync_copy(rows_v, dst.at[eidx_v.at[j, 0]], add=True)

        for half in range(4):
            pltpu.sync_copy(eidx_hbm.at[pl.ds(j0_tile + half * JH, JSTAGE)],
                            eidx_v)
            pltpu.sync_copy(evals_hbm.at[pl.ds(j0_tile + half * JH, JSTAGE)],
                            vals_v)
            pltpu.async_copy(src.at[eidx_v.at[0, 1]], rows_a, gsem_a)

            def body(jj, carry):
                ja = 2 * jj
                jb = 2 * jj + 1
                gb = pltpu.async_copy(src.at[eidx_v.at[jb, 1]], rows_b, gsem_b)
                pltpu.make_async_copy(src.at[eidx_v.at[ja, 1]], rows_a,
                                      gsem_a).wait()
                process(ja, rows_a)
                pltpu.async_copy(src.at[eidx_v.at[ja + 2, 1]], rows_a, gsem_a)
                gb.wait()
                process(jb, rows_b)
                return carry
            lax.fori_loop(0, JH // 2, body, 0)
            pltpu.make_async_copy(src.at[eidx_v.at[0, 1]], rows_a,
                                  gsem_a).wait()

    run_layer(src_sh, acc_sh)
    plsc.subcore_barrier()
    for z in range(ROWS_PER_TILE // ZROWS):
        pltpu.sync_copy(zero_v, src_sh.at[pl.ds(r0 + z * ZROWS, ZROWS)])
    plsc.subcore_barrier()
    run_layer(acc_sh, src_sh)
    plsc.subcore_barrier()

    pltpu.sync_copy(src_sh.at[pl.ds(r0, ROWS_PER_TILE)],
                    out_hbm.at[c, pl.ds(r0, ROWS_PER_TILE)])


_spmm2 = _build()(_spmm2_body)


def kernel(x, edge_index, values):
    xs = jnp.concatenate([x[:, :HALF], x[:, HALF:]], axis=0)
    pad = E_PAD - N_EDGES
    row = jnp.pad(edge_index[0], (0, pad)).reshape(NCHUNK, REAL)
    col = jnp.pad(edge_index[1], (0, pad)).reshape(NCHUNK, REAL)
    val = jnp.pad(values, (0, pad)).reshape(NCHUNK, REAL)
    zero_lane_i = jnp.zeros((NCHUNK, 1), jnp.int32)
    zero_lane_f = jnp.zeros((NCHUNK, 1), jnp.float32)
    row = jnp.concatenate([zero_lane_i, row], axis=1)[:, None, :]
    col = jnp.concatenate([zero_lane_i, col], axis=1)[:, None, :]
    eidx = jnp.concatenate([row, col], axis=1)
    evals = jnp.concatenate([zero_lane_f, val], axis=1)
    eidx = jnp.pad(eidx, ((0, NC_OUT - NCHUNK), (0, 0), (0, 0)))
    evals = jnp.pad(evals, ((0, NC_OUT - NCHUNK), (0, 0)))
    o = _spmm2(xs, eidx, evals)
    return jnp.concatenate([o[0], o[1]], axis=1)
